# Initial kernel scaffold; baseline (speedup 1.0000x reference)
import jax
import jax.numpy as jnp
from jax.experimental import pallas as pl


def kernel(x, emb_table, w2, b2, w3, b3, w4, b4):
    raise NotImplementedError("write your pallas kernel here")


# speedup vs baseline: 4.4524x; 4.4524x over previous
import functools

import jax
import jax.numpy as jnp
from jax.experimental import pallas as pl
from jax.experimental.pallas import tpu as pltpu

_VOCAB = 128
_EMBED = 30
_F = 50
_C = 24
_NTAP = 4
_NCOL = 256
_NEG = -1e30


def _tables_kernel(emb_ref, wt_ref, t_ref):
    blocks = []
    for j in range(_NTAP):
        blocks.append(
            jax.lax.dot_general(
                emb_ref[...], wt_ref[j],
                dimension_numbers=(((1,), (0,)), ((), ())),
                preferred_element_type=jnp.float32,
                precision=jax.lax.Precision.HIGHEST,
            ))
    t_ref[...] = jnp.concatenate(blocks, axis=0)


def _main_kernel(ids_ref, t_ref, mpb_ref, out_ref):
    rblk = ids_ref.shape[0]
    ids = ids_ref[...]
    iota = jax.lax.broadcasted_iota(jnp.int32, (rblk, _VOCAB), 1)
    o0 = jnp.where(iota == ids, 1.0, 0.0)
    parts = [o0]
    for j in range(1, _NTAP):
        parts.append(pltpu.roll(o0, rblk - j, axis=0))
    lhs = jnp.concatenate(parts, axis=1)
    y = jax.lax.dot_general(
        lhs, t_ref[...], dimension_numbers=(((1,), (0,)), ((), ())),
        preferred_element_type=jnp.float32)
    w = rblk // _C
    y3 = y.reshape(w, _C, _NCOL) + mpb_ref[...][None, :, :]
    pooled = jnp.max(y3, axis=1)
    out_ref[...] = jnp.maximum(pooled, 0.0)[:, :3 * _F]


@jax.jit
def kernel(x, emb_table, w2, b2, w3, b3, w4, b4):
    B, S, C = x.shape
    n_words = B * S
    n_rows = n_words * C

    ws = {2: w2, 3: w3, 4: w4}
    bs = {2: b2, 3: b3, 4: b4}
    zeros_tap = jnp.zeros((_EMBED, _F), jnp.float32)
    wt_rows = []
    for j in range(_NTAP):
        cols = [ws[k][:, :, j].T if j < k else zeros_tap for k in (2, 3, 4)]
        wt_rows.append(jnp.pad(jnp.concatenate(cols, axis=1),
                               ((0, 0), (0, _NCOL - 3 * _F))))
    wt = jnp.stack(wt_rows)

    t_cat = pl.pallas_call(
        _tables_kernel,
        out_shape=jax.ShapeDtypeStruct((_NTAP * _VOCAB, _NCOL), jnp.float32),
    )(emb_table, wt)

    pos = jnp.arange(_C, dtype=jnp.int32)[:, None]
    kcol = jnp.concatenate([
        jnp.full((_F,), k, jnp.int32) for k in (2, 3, 4)
    ] + [jnp.full((_NCOL - 3 * _F,), 127, jnp.int32)])
    bias_row = jnp.concatenate(
        [bs[k] for k in (2, 3, 4)] +
        [jnp.zeros((_NCOL - 3 * _F,), jnp.float32)])
    mpb = jnp.where(pos <= _C - kcol[None, :], bias_row[None, :], _NEG)

    ids2d = x.reshape(n_rows, 1)

    words_per_blk = 128
    rblk = words_per_blk * _C
    n_blocks = n_words // words_per_blk

    out = pl.pallas_call(
        _main_kernel,
        grid=(n_blocks,),
        in_specs=[
            pl.BlockSpec((rblk, 1), lambda i: (i, 0)),
            pl.BlockSpec((_NTAP * _VOCAB, _NCOL), lambda i: (0, 0)),
            pl.BlockSpec((_C, _NCOL), lambda i: (0, 0)),
        ],
        out_specs=pl.BlockSpec((words_per_blk, 3 * _F), lambda i: (i, 0)),
        out_shape=jax.ShapeDtypeStruct((n_words, 3 * _F), jnp.float32),
        compiler_params=pltpu.CompilerParams(
            dimension_semantics=("parallel",)),
    )(ids2d, t_cat, mpb)

    return out.reshape(B, S, 3 * _F)
